# baseline (device time: 212283 ns/iter reference)
import jax
import jax.numpy as jnp
from jax import lax
from jax.experimental import pallas as pl
from jax.experimental.pallas import tpu as pltpu

N_DEV = 16
B, SQ, SKV, HQ, DH = 2, 512, 512, 128, 64
H_LOC = HQ // N_DEV
D_MODEL = 768
ROWS = B * SQ
CHUNK = ROWS // N_DEV
BLK = 64

_SHIFT = {1: 0, 2: 1, 4: 2, 8: 3}
_RS_DIMS = [1, 4, 2, 8]
_RS_ROWS = [512, 256, 128, 64]
_RS_OFF = [0, 512, 768, 896]

SEQ_BLK = 64


def _kv_slice_body(k_win, v_win, ko_ref, vo_ref):
    h0 = lax.axis_index("i") * H_LOC
    ko_ref[...] = k_win[:, :, pl.ds(h0, H_LOC), :]
    vo_ref[...] = v_win[:, :, pl.ds(h0, H_LOC), :]


def _slice_kv(K_ext, V_ext):
    return pl.pallas_call(
        _kv_slice_body,
        grid=(B, SKV // SEQ_BLK),
        in_specs=[
            pl.BlockSpec((1, SEQ_BLK, HQ, DH), lambda i, j: (i, j, 0, 0)),
            pl.BlockSpec((1, SEQ_BLK, HQ, DH), lambda i, j: (i, j, 0, 0)),
        ],
        out_specs=[
            pl.BlockSpec((1, SEQ_BLK, H_LOC, DH), lambda i, j: (i, j, 0, 0)),
            pl.BlockSpec((1, SEQ_BLK, H_LOC, DH), lambda i, j: (i, j, 0, 0)),
        ],
        out_shape=[
            jax.ShapeDtypeStruct((B, SKV, H_LOC, DH), jnp.float32),
            jax.ShapeDtypeStruct((B, SKV, H_LOC, DH), jnp.float32),
        ],
    )(K_ext, V_ext)


def _body(x_ref, wq_ref, k_ref, v_ref, wo_ref, out_ref,
          ctx_ref, rs_send, rs_recv, ag_buf,
          rs_send_sems, rs_recv_sems, ag_send_sems, ag_recv_sems):
    my = lax.axis_index("i")

    barrier = pltpu.get_barrier_semaphore()
    for d in _RS_DIMS:
        pl.semaphore_signal(
            barrier, inc=1,
            device_id=(my ^ d,), device_id_type=pl.DeviceIdType.MESH,
        )
    pl.semaphore_wait(barrier, 4)

    qb = lax.broadcasted_iota(jnp.int32, (SQ, SKV), 0) // BLK
    kb = lax.broadcasted_iota(jnp.int32, (SQ, SKV), 1) // BLK
    mask = (qb == kb) | (kb == 0) | (((qb + kb) % 3) == 0)

    bit0 = my & 1
    pending = []

    for j in range(2):
        b = bit0 ^ (1 - j)
        xb = x_ref[pl.ds(b, 1)].reshape(SQ, D_MODEL)
        q_all = jnp.dot(xb, wq_ref[...], preferred_element_type=jnp.float32)
        for h in range(H_LOC):
            q = q_all[:, h * DH:(h + 1) * DH]
            k = k_ref[pl.ds(b, 1), :, h, :].reshape(SKV, DH)
            v = v_ref[pl.ds(b, 1), :, h, :].reshape(SKV, DH)
            s = lax.dot_general(
                q, k, (((1,), (1,)), ((), ())),
                preferred_element_type=jnp.float32,
            ) * 0.125
            s = jnp.where(mask, s, -1e9)
            m = jnp.max(s, axis=1, keepdims=True)
            w = jnp.exp(s - m)
            w = w / jnp.sum(w, axis=1, keepdims=True)
            ctx_ref[:, h * DH:(h + 1) * DH] = jnp.dot(
                w, v, preferred_element_type=jnp.float32)
        partial = jnp.dot(ctx_ref[...], wo_ref[...],
                          preferred_element_type=jnp.float32)
        out_ref[pl.ds(b * SQ, SQ), :] = partial
        if j == 0:
            rs_send[pl.ds(0, 512), :] = partial.astype(jnp.bfloat16)
            rdma = pltpu.make_async_remote_copy(
                src_ref=rs_send.at[pl.ds(0, 512), :],
                dst_ref=rs_recv.at[pl.ds(0, 512), :],
                send_sem=rs_send_sems.at[0],
                recv_sem=rs_recv_sems.at[0],
                device_id=(my ^ 1,),
                device_id_type=pl.DeviceIdType.MESH,
            )
            rdma.start()
            pending.append(rdma)
            rdma0 = rdma

    lo = bit0 * 8
    rdma0.wait_recv()
    out_ref[pl.ds(lo * CHUNK, 512), :] = (
        out_ref[pl.ds(lo * CHUNK, 512), :]
        + rs_recv[pl.ds(0, 512), :].astype(jnp.float32))

    sz = 8
    for step in range(1, 4):
        d = _RS_DIMS[step]
        szh = sz // 2
        rows = _RS_ROWS[step]
        off = _RS_OFF[step]
        bit = (my >> _SHIFT[d]) & 1
        send_lo = lo + (1 - bit) * szh
        keep_lo = lo + bit * szh
        rs_send[pl.ds(off, rows), :] = (
            out_ref[pl.ds(send_lo * CHUNK, rows), :].astype(jnp.bfloat16))
        rdma = pltpu.make_async_remote_copy(
            src_ref=rs_send.at[pl.ds(off, rows), :],
            dst_ref=rs_recv.at[pl.ds(off, rows), :],
            send_sem=rs_send_sems.at[step],
            recv_sem=rs_recv_sems.at[step],
            device_id=(my ^ d,),
            device_id_type=pl.DeviceIdType.MESH,
        )
        rdma.start()
        pending.append(rdma)
        rdma.wait_recv()
        out_ref[pl.ds(keep_lo * CHUNK, rows), :] = (
            out_ref[pl.ds(keep_lo * CHUNK, rows), :]
            + rs_recv[pl.ds(off, rows), :].astype(jnp.float32))
        lo = keep_lo
        sz = szh

    ag_buf[pl.ds(lo * CHUNK, CHUNK), :] = (
        out_ref[pl.ds(lo * CHUNK, CHUNK), :].astype(jnp.bfloat16))
    sz = 1
    for step, d in enumerate(reversed(_RS_DIMS)):
        bit = (my >> _SHIFT[d]) & 1
        rows = sz * CHUNK
        rdma = pltpu.make_async_remote_copy(
            src_ref=ag_buf.at[pl.ds(lo * CHUNK, rows), :],
            dst_ref=ag_buf.at[pl.ds(lo * CHUNK, rows), :],
            send_sem=ag_send_sems.at[step],
            recv_sem=ag_recv_sems.at[step],
            device_id=(my ^ d,),
            device_id_type=pl.DeviceIdType.MESH,
        )
        rdma.start()
        pending.append(rdma)
        rdma.wait_recv()
        lo = lo - bit * sz
        sz *= 2

    out_ref[...] = ag_buf[...].astype(jnp.float32)

    for rdma in pending:
        rdma.wait_send()


def kernel(x, Wq, K_ext, V_ext, Wo):
    k_loc, v_loc = _slice_kv(K_ext, V_ext)
    out = pl.pallas_call(
        _body,
        out_shape=jax.ShapeDtypeStruct((ROWS, D_MODEL), jnp.float32),
        in_specs=[
            pl.BlockSpec(memory_space=pltpu.VMEM),
            pl.BlockSpec(memory_space=pltpu.VMEM),
            pl.BlockSpec(memory_space=pltpu.VMEM),
            pl.BlockSpec(memory_space=pltpu.VMEM),
            pl.BlockSpec(memory_space=pltpu.VMEM),
        ],
        out_specs=pl.BlockSpec(memory_space=pltpu.VMEM),
        scratch_shapes=[
            pltpu.VMEM((SQ, H_LOC * DH), jnp.float32),
            pltpu.VMEM((960, D_MODEL), jnp.bfloat16),
            pltpu.VMEM((960, D_MODEL), jnp.bfloat16),
            pltpu.VMEM((ROWS, D_MODEL), jnp.bfloat16),
            pltpu.SemaphoreType.DMA((4,)),
            pltpu.SemaphoreType.DMA((4,)),
            pltpu.SemaphoreType.DMA((4,)),
            pltpu.SemaphoreType.DMA((4,)),
        ],
        compiler_params=pltpu.CompilerParams(
            collective_id=0, vmem_limit_bytes=60 * 1024 * 1024),
    )(x, Wq, k_loc, v_loc, Wo)
    return out.reshape(B, SQ, D_MODEL)


# device time: 202736 ns/iter; 1.0471x vs baseline; 1.0471x over previous
import jax
import jax.numpy as jnp
from jax import lax
from jax.experimental import pallas as pl
from jax.experimental.pallas import tpu as pltpu

N_DEV = 16
B, SQ, SKV, HQ, DH = 2, 512, 512, 128, 64
H_LOC = HQ // N_DEV
D_MODEL = 768
ROWS = B * SQ
CHUNK = ROWS // N_DEV
BLK = 64

_SHIFT = {1: 0, 2: 1, 4: 2, 8: 3}
_RS_DIMS = [1, 4, 2, 8]
_RS_ROWS = [512, 256, 128, 64]
_RS_OFF = [0, 512, 768, 896]

SEQ_BLK = 64


N_SBLK = SKV // SEQ_BLK


def _body(x_ref, wq_ref, k_hbm, v_hbm, wo_ref, out_ref,
          stage_k, stage_v, k_ref, v_ref, ctx_ref, rs_send, rs_recv, ag_buf,
          kv_sems, rs_send_sems, rs_recv_sems, ag_send_sems, ag_recv_sems):
    my = lax.axis_index("i")
    h0 = my * H_LOC

    barrier = pltpu.get_barrier_semaphore()
    for d in _RS_DIMS:
        pl.semaphore_signal(
            barrier, inc=1,
            device_id=(my ^ d,), device_id_type=pl.DeviceIdType.MESH,
        )

    def _issue(i):
        slot = i % 2
        b_i, s_i = divmod(i, N_SBLK)
        s0 = s_i * SEQ_BLK
        ck = pltpu.make_async_copy(
            k_hbm.at[b_i, pl.ds(s0, SEQ_BLK), :, :], stage_k.at[slot],
            kv_sems.at[2 * slot])
        cv = pltpu.make_async_copy(
            v_hbm.at[b_i, pl.ds(s0, SEQ_BLK), :, :], stage_v.at[slot],
            kv_sems.at[2 * slot + 1])
        ck.start()
        cv.start()
        return ck, cv

    inflight = {0: _issue(0), 1: _issue(1)}
    for i in range(B * N_SBLK):
        ck, cv = inflight.pop(i % 2)
        ck.wait()
        cv.wait()
        b_i, s_i = divmod(i, N_SBLK)
        s0 = s_i * SEQ_BLK
        k_ref[b_i, pl.ds(s0, SEQ_BLK), :, :] = (
            stage_k[i % 2, :, pl.ds(h0, H_LOC), :])
        v_ref[b_i, pl.ds(s0, SEQ_BLK), :, :] = (
            stage_v[i % 2, :, pl.ds(h0, H_LOC), :])
        if i + 2 < B * N_SBLK:
            inflight[i % 2] = _issue(i + 2)

    pl.semaphore_wait(barrier, 4)

    qb = lax.broadcasted_iota(jnp.int32, (SQ, SKV), 0) // BLK
    kb = lax.broadcasted_iota(jnp.int32, (SQ, SKV), 1) // BLK
    mask = (qb == kb) | (kb == 0) | (((qb + kb) % 3) == 0)

    bit0 = my & 1
    pending = []

    for j in range(2):
        b = bit0 ^ (1 - j)
        xb = x_ref[pl.ds(b, 1)].reshape(SQ, D_MODEL)
        q_all = jnp.dot(xb, wq_ref[...], preferred_element_type=jnp.float32)
        for h in range(H_LOC):
            q = q_all[:, h * DH:(h + 1) * DH]
            k = k_ref[pl.ds(b, 1), :, h, :].reshape(SKV, DH)
            v = v_ref[pl.ds(b, 1), :, h, :].reshape(SKV, DH)
            s = lax.dot_general(
                q, k, (((1,), (1,)), ((), ())),
                preferred_element_type=jnp.float32,
            ) * 0.125
            s = jnp.where(mask, s, -1e9)
            m = jnp.max(s, axis=1, keepdims=True)
            w = jnp.exp(s - m)
            w = w / jnp.sum(w, axis=1, keepdims=True)
            ctx_ref[:, h * DH:(h + 1) * DH] = jnp.dot(
                w, v, preferred_element_type=jnp.float32)
        partial = jnp.dot(ctx_ref[...], wo_ref[...],
                          preferred_element_type=jnp.float32)
        out_ref[pl.ds(b * SQ, SQ), :] = partial
        if j == 0:
            rs_send[pl.ds(0, 512), :] = partial.astype(jnp.bfloat16)
            rdma = pltpu.make_async_remote_copy(
                src_ref=rs_send.at[pl.ds(0, 512), :],
                dst_ref=rs_recv.at[pl.ds(0, 512), :],
                send_sem=rs_send_sems.at[0],
                recv_sem=rs_recv_sems.at[0],
                device_id=(my ^ 1,),
                device_id_type=pl.DeviceIdType.MESH,
            )
            rdma.start()
            pending.append(rdma)
            rdma0 = rdma

    lo = bit0 * 8
    rdma0.wait_recv()
    out_ref[pl.ds(lo * CHUNK, 512), :] = (
        out_ref[pl.ds(lo * CHUNK, 512), :]
        + rs_recv[pl.ds(0, 512), :].astype(jnp.float32))

    sz = 8
    for step in range(1, 4):
        d = _RS_DIMS[step]
        szh = sz // 2
        rows = _RS_ROWS[step]
        off = _RS_OFF[step]
        bit = (my >> _SHIFT[d]) & 1
        send_lo = lo + (1 - bit) * szh
        keep_lo = lo + bit * szh
        rs_send[pl.ds(off, rows), :] = (
            out_ref[pl.ds(send_lo * CHUNK, rows), :].astype(jnp.bfloat16))
        rdma = pltpu.make_async_remote_copy(
            src_ref=rs_send.at[pl.ds(off, rows), :],
            dst_ref=rs_recv.at[pl.ds(off, rows), :],
            send_sem=rs_send_sems.at[step],
            recv_sem=rs_recv_sems.at[step],
            device_id=(my ^ d,),
            device_id_type=pl.DeviceIdType.MESH,
        )
        rdma.start()
        pending.append(rdma)
        rdma.wait_recv()
        out_ref[pl.ds(keep_lo * CHUNK, rows), :] = (
            out_ref[pl.ds(keep_lo * CHUNK, rows), :]
            + rs_recv[pl.ds(off, rows), :].astype(jnp.float32))
        lo = keep_lo
        sz = szh

    ag_buf[pl.ds(lo * CHUNK, CHUNK), :] = (
        out_ref[pl.ds(lo * CHUNK, CHUNK), :].astype(jnp.bfloat16))
    sz = 1
    for step, d in enumerate(reversed(_RS_DIMS)):
        bit = (my >> _SHIFT[d]) & 1
        rows = sz * CHUNK
        rdma = pltpu.make_async_remote_copy(
            src_ref=ag_buf.at[pl.ds(lo * CHUNK, rows), :],
            dst_ref=ag_buf.at[pl.ds(lo * CHUNK, rows), :],
            send_sem=ag_send_sems.at[step],
            recv_sem=ag_recv_sems.at[step],
            device_id=(my ^ d,),
            device_id_type=pl.DeviceIdType.MESH,
        )
        rdma.start()
        pending.append(rdma)
        rdma.wait_recv()
        lo = lo - bit * sz
        sz *= 2

    out_ref[...] = ag_buf[...].astype(jnp.float32)

    for rdma in pending:
        rdma.wait_send()


def kernel(x, Wq, K_ext, V_ext, Wo):
    out = pl.pallas_call(
        _body,
        out_shape=jax.ShapeDtypeStruct((ROWS, D_MODEL), jnp.float32),
        in_specs=[
            pl.BlockSpec(memory_space=pltpu.VMEM),
            pl.BlockSpec(memory_space=pltpu.VMEM),
            pl.BlockSpec(memory_space=pltpu.MemorySpace.HBM),
            pl.BlockSpec(memory_space=pltpu.MemorySpace.HBM),
            pl.BlockSpec(memory_space=pltpu.VMEM),
        ],
        out_specs=pl.BlockSpec(memory_space=pltpu.VMEM),
        scratch_shapes=[
            pltpu.VMEM((2, SEQ_BLK, HQ, DH), jnp.float32),
            pltpu.VMEM((2, SEQ_BLK, HQ, DH), jnp.float32),
            pltpu.VMEM((B, SKV, H_LOC, DH), jnp.float32),
            pltpu.VMEM((B, SKV, H_LOC, DH), jnp.float32),
            pltpu.VMEM((SQ, H_LOC * DH), jnp.float32),
            pltpu.VMEM((960, D_MODEL), jnp.bfloat16),
            pltpu.VMEM((960, D_MODEL), jnp.bfloat16),
            pltpu.VMEM((ROWS, D_MODEL), jnp.bfloat16),
            pltpu.SemaphoreType.DMA((4,)),
            pltpu.SemaphoreType.DMA((4,)),
            pltpu.SemaphoreType.DMA((4,)),
            pltpu.SemaphoreType.DMA((4,)),
            pltpu.SemaphoreType.DMA((4,)),
        ],
        compiler_params=pltpu.CompilerParams(
            collective_id=0, vmem_limit_bytes=60 * 1024 * 1024),
    )(x, Wq, K_ext, V_ext, Wo)
    return out.reshape(B, SQ, D_MODEL)


# device time: 160055 ns/iter; 1.3263x vs baseline; 1.2667x over previous
import jax
import jax.numpy as jnp
from jax import lax
from jax.experimental import pallas as pl
from jax.experimental.pallas import tpu as pltpu

N_DEV = 16
B, SQ, SKV, HQ, DH = 2, 512, 512, 128, 64
H_LOC = HQ // N_DEV
D_MODEL = 768
ROWS = B * SQ
CHUNK = ROWS // N_DEV
BLK = 64

_SHIFT = {1: 0, 2: 1, 4: 2, 8: 3}
_RS_DIMS = [1, 4, 2, 8]
_RS_ROWS = [512, 256, 128, 64]
_RS_OFF = [0, 512, 768, 896]


def _body(x_ref, wq_ref, k_ref, v_ref, wo_ref, out_ref,
          ctx_ref, rs_send, rs_recv, ag_buf,
          rs_send_sems, rs_recv_sems, ag_send_sems, ag_recv_sems):
    my = lax.axis_index("i")

    barrier = pltpu.get_barrier_semaphore()
    for d in _RS_DIMS:
        pl.semaphore_signal(
            barrier, inc=1,
            device_id=(my ^ d,), device_id_type=pl.DeviceIdType.MESH,
        )
    pl.semaphore_wait(barrier, 4)

    qb = lax.broadcasted_iota(jnp.int32, (SQ, SKV), 0) // BLK
    kb = lax.broadcasted_iota(jnp.int32, (SQ, SKV), 1) // BLK
    mask = (qb == kb) | (kb == 0) | (((qb + kb) % 3) == 0)

    bit0 = my & 1
    pending = []

    for j in range(2):
        b = bit0 ^ (1 - j)
        xb = x_ref[pl.ds(b, 1)].reshape(SQ, D_MODEL)
        kb_all = k_ref[pl.ds(b, 1), :, pl.ds(my * H_LOC * DH, H_LOC * DH)
                       ].reshape(SKV, H_LOC * DH)
        vb_all = v_ref[pl.ds(b, 1), :, pl.ds(my * H_LOC * DH, H_LOC * DH)
                       ].reshape(SKV, H_LOC * DH)
        q_all = jnp.dot(xb, wq_ref[...], preferred_element_type=jnp.float32)
        for h in range(H_LOC):
            q = q_all[:, h * DH:(h + 1) * DH].astype(jnp.bfloat16)
            k = kb_all[:, h * DH:(h + 1) * DH]
            v = vb_all[:, h * DH:(h + 1) * DH]
            s = lax.dot_general(
                q, k, (((1,), (1,)), ((), ())),
                preferred_element_type=jnp.float32,
            ) * 0.125
            s = jnp.where(mask, s, -1e9)
            m = jnp.max(s, axis=1, keepdims=True)
            w = jnp.exp(s - m)
            w = (w / jnp.sum(w, axis=1, keepdims=True)).astype(jnp.bfloat16)
            ctx_ref[:, h * DH:(h + 1) * DH] = jnp.dot(
                w, v, preferred_element_type=jnp.float32)
        partial = jnp.dot(ctx_ref[...], wo_ref[...],
                          preferred_element_type=jnp.float32)
        out_ref[pl.ds(b * SQ, SQ), :] = partial
        if j == 0:
            rs_send[pl.ds(0, 512), :] = partial.astype(jnp.bfloat16)
            rdma = pltpu.make_async_remote_copy(
                src_ref=rs_send.at[pl.ds(0, 512), :],
                dst_ref=rs_recv.at[pl.ds(0, 512), :],
                send_sem=rs_send_sems.at[0],
                recv_sem=rs_recv_sems.at[0],
                device_id=(my ^ 1,),
                device_id_type=pl.DeviceIdType.MESH,
            )
            rdma.start()
            pending.append(rdma)
            rdma0 = rdma

    lo = bit0 * 8
    rdma0.wait_recv()
    out_ref[pl.ds(lo * CHUNK, 512), :] = (
        out_ref[pl.ds(lo * CHUNK, 512), :]
        + rs_recv[pl.ds(0, 512), :].astype(jnp.float32))

    sz = 8
    for step in range(1, 4):
        d = _RS_DIMS[step]
        szh = sz // 2
        rows = _RS_ROWS[step]
        off = _RS_OFF[step]
        bit = (my >> _SHIFT[d]) & 1
        send_lo = lo + (1 - bit) * szh
        keep_lo = lo + bit * szh
        rs_send[pl.ds(off, rows), :] = (
            out_ref[pl.ds(send_lo * CHUNK, rows), :].astype(jnp.bfloat16))
        rdma = pltpu.make_async_remote_copy(
            src_ref=rs_send.at[pl.ds(off, rows), :],
            dst_ref=rs_recv.at[pl.ds(off, rows), :],
            send_sem=rs_send_sems.at[step],
            recv_sem=rs_recv_sems.at[step],
            device_id=(my ^ d,),
            device_id_type=pl.DeviceIdType.MESH,
        )
        rdma.start()
        pending.append(rdma)
        rdma.wait_recv()
        out_ref[pl.ds(keep_lo * CHUNK, rows), :] = (
            out_ref[pl.ds(keep_lo * CHUNK, rows), :]
            + rs_recv[pl.ds(off, rows), :].astype(jnp.float32))
        lo = keep_lo
        sz = szh

    ag_buf[pl.ds(lo * CHUNK, CHUNK), :] = (
        out_ref[pl.ds(lo * CHUNK, CHUNK), :].astype(jnp.bfloat16))
    sz = 1
    for step, d in enumerate(reversed(_RS_DIMS)):
        bit = (my >> _SHIFT[d]) & 1
        rows = sz * CHUNK
        rdma = pltpu.make_async_remote_copy(
            src_ref=ag_buf.at[pl.ds(lo * CHUNK, rows), :],
            dst_ref=ag_buf.at[pl.ds(lo * CHUNK, rows), :],
            send_sem=ag_send_sems.at[step],
            recv_sem=ag_recv_sems.at[step],
            device_id=(my ^ d,),
            device_id_type=pl.DeviceIdType.MESH,
        )
        rdma.start()
        pending.append(rdma)
        rdma.wait_recv()
        lo = lo - bit * sz
        sz *= 2

    out_ref[...] = ag_buf[...].astype(jnp.float32)

    for rdma in pending:
        rdma.wait_send()


def kernel(x, Wq, K_ext, V_ext, Wo):
    K2 = K_ext.astype(jnp.bfloat16).reshape(B, SKV, HQ * DH)
    V2 = V_ext.astype(jnp.bfloat16).reshape(B, SKV, HQ * DH)
    out = pl.pallas_call(
        _body,
        out_shape=jax.ShapeDtypeStruct((ROWS, D_MODEL), jnp.float32),
        in_specs=[
            pl.BlockSpec(memory_space=pltpu.VMEM),
            pl.BlockSpec(memory_space=pltpu.VMEM),
            pl.BlockSpec(memory_space=pltpu.VMEM),
            pl.BlockSpec(memory_space=pltpu.VMEM),
            pl.BlockSpec(memory_space=pltpu.VMEM),
        ],
        out_specs=pl.BlockSpec(memory_space=pltpu.VMEM),
        scratch_shapes=[
            pltpu.VMEM((SQ, H_LOC * DH), jnp.float32),
            pltpu.VMEM((960, D_MODEL), jnp.bfloat16),
            pltpu.VMEM((960, D_MODEL), jnp.bfloat16),
            pltpu.VMEM((ROWS, D_MODEL), jnp.bfloat16),
            pltpu.SemaphoreType.DMA((4,)),
            pltpu.SemaphoreType.DMA((4,)),
            pltpu.SemaphoreType.DMA((4,)),
            pltpu.SemaphoreType.DMA((4,)),
        ],
        compiler_params=pltpu.CompilerParams(
            collective_id=0, vmem_limit_bytes=60 * 1024 * 1024),
    )(x, Wq, K2, V2, Wo)
    return out.reshape(B, SQ, D_MODEL)


# device time: 134178 ns/iter; 1.5821x vs baseline; 1.1929x over previous
import jax
import jax.numpy as jnp
from jax import lax
from jax.experimental import pallas as pl
from jax.experimental.pallas import tpu as pltpu

N_DEV = 16
B, SQ, SKV, HQ, DH = 2, 512, 512, 128, 64
H_LOC = HQ // N_DEV
D_MODEL = 768
ROWS = B * SQ
CHUNK = ROWS // N_DEV
BLK = 64

_SHIFT = {1: 0, 2: 1, 4: 2, 8: 3}
_RS_DIMS = [1, 4, 2, 8]
_RS_ROWS = [512, 256, 128, 64]
_RS_OFF = [0, 512, 768, 896]


def _body(x_ref, wq_ref, k_ref, v_ref, wo_ref, out_ref,
          ctx_ref, rs_send, rs_recv, ag_buf,
          rs_send_sems, rs_recv_sems, ag_send_sems, ag_recv_sems):
    my = lax.axis_index("i")

    barrier = pltpu.get_barrier_semaphore()
    for d in _RS_DIMS:
        pl.semaphore_signal(
            barrier, inc=1,
            device_id=(my ^ d,), device_id_type=pl.DeviceIdType.MESH,
        )
    pl.semaphore_wait(barrier, 4)

    qb = lax.broadcasted_iota(jnp.int32, (SQ, SKV), 0) // BLK
    kb = lax.broadcasted_iota(jnp.int32, (SQ, SKV), 1) // BLK
    mask = (qb == kb) | (kb == 0) | (((qb + kb) % 3) == 0)

    bit0 = my & 1
    pending = []

    for j in range(2):
        b = bit0 ^ (1 - j)
        xb = x_ref[pl.ds(b, 1)].reshape(SQ, D_MODEL)
        kb_all = k_ref[pl.ds(b, 1)].reshape(SKV, H_LOC * DH)
        vb_all = v_ref[pl.ds(b, 1)].reshape(SKV, H_LOC * DH)
        q_all = jnp.dot(xb, wq_ref[...], preferred_element_type=jnp.float32)
        for h in range(H_LOC):
            q = q_all[:, h * DH:(h + 1) * DH].astype(jnp.bfloat16)
            k = kb_all[:, h * DH:(h + 1) * DH]
            v = vb_all[:, h * DH:(h + 1) * DH]
            s = lax.dot_general(
                q, k, (((1,), (1,)), ((), ())),
                preferred_element_type=jnp.float32,
            ) * 0.125
            s = jnp.where(mask, s, -1e9)
            m = jnp.max(s, axis=1, keepdims=True)
            w = jnp.exp(s - m)
            w = (w / jnp.sum(w, axis=1, keepdims=True)).astype(jnp.bfloat16)
            ctx_ref[:, h * DH:(h + 1) * DH] = jnp.dot(
                w, v, preferred_element_type=jnp.float32)
        partial = jnp.dot(ctx_ref[...].astype(jnp.bfloat16), wo_ref[...],
                          preferred_element_type=jnp.float32)
        out_ref[pl.ds(b * SQ, SQ), :] = partial
        if j == 0:
            rs_send[pl.ds(0, 512), :] = partial.astype(jnp.bfloat16)
            rdma = pltpu.make_async_remote_copy(
                src_ref=rs_send.at[pl.ds(0, 512), :],
                dst_ref=rs_recv.at[pl.ds(0, 512), :],
                send_sem=rs_send_sems.at[0],
                recv_sem=rs_recv_sems.at[0],
                device_id=(my ^ 1,),
                device_id_type=pl.DeviceIdType.MESH,
            )
            rdma.start()
            pending.append(rdma)
            rdma0 = rdma

    lo = bit0 * 8
    rdma0.wait_recv()
    out_ref[pl.ds(lo * CHUNK, 512), :] = (
        out_ref[pl.ds(lo * CHUNK, 512), :]
        + rs_recv[pl.ds(0, 512), :].astype(jnp.float32))

    sz = 8
    for step in range(1, 4):
        d = _RS_DIMS[step]
        szh = sz // 2
        rows = _RS_ROWS[step]
        off = _RS_OFF[step]
        bit = (my >> _SHIFT[d]) & 1
        send_lo = lo + (1 - bit) * szh
        keep_lo = lo + bit * szh
        rs_send[pl.ds(off, rows), :] = (
            out_ref[pl.ds(send_lo * CHUNK, rows), :].astype(jnp.bfloat16))
        rdma = pltpu.make_async_remote_copy(
            src_ref=rs_send.at[pl.ds(off, rows), :],
            dst_ref=rs_recv.at[pl.ds(off, rows), :],
            send_sem=rs_send_sems.at[step],
            recv_sem=rs_recv_sems.at[step],
            device_id=(my ^ d,),
            device_id_type=pl.DeviceIdType.MESH,
        )
        rdma.start()
        pending.append(rdma)
        rdma.wait_recv()
        out_ref[pl.ds(keep_lo * CHUNK, rows), :] = (
            out_ref[pl.ds(keep_lo * CHUNK, rows), :]
            + rs_recv[pl.ds(off, rows), :].astype(jnp.float32))
        lo = keep_lo
        sz = szh

    ag_buf[pl.ds(lo * CHUNK, CHUNK), :] = (
        out_ref[pl.ds(lo * CHUNK, CHUNK), :].astype(jnp.bfloat16))
    sz = 1
    for step, d in enumerate(reversed(_RS_DIMS)):
        bit = (my >> _SHIFT[d]) & 1
        rows = sz * CHUNK
        rdma = pltpu.make_async_remote_copy(
            src_ref=ag_buf.at[pl.ds(lo * CHUNK, rows), :],
            dst_ref=ag_buf.at[pl.ds(lo * CHUNK, rows), :],
            send_sem=ag_send_sems.at[step],
            recv_sem=ag_recv_sems.at[step],
            device_id=(my ^ d,),
            device_id_type=pl.DeviceIdType.MESH,
        )
        rdma.start()
        pending.append(rdma)
        rdma.wait_recv()
        lo = lo - bit * sz
        sz *= 2

    out_ref[...] = ag_buf[...].astype(jnp.float32)

    for rdma in pending:
        rdma.wait_send()


def kernel(x, Wq, K_ext, V_ext, Wo):
    my = lax.axis_index("i")
    K2 = lax.dynamic_slice_in_dim(K_ext, my * H_LOC, H_LOC, axis=2
                                  ).reshape(B, SKV, H_LOC * DH
                                            ).astype(jnp.bfloat16)
    V2 = lax.dynamic_slice_in_dim(V_ext, my * H_LOC, H_LOC, axis=2
                                  ).reshape(B, SKV, H_LOC * DH
                                            ).astype(jnp.bfloat16)
    x16 = x.astype(jnp.bfloat16)
    Wq16 = Wq.astype(jnp.bfloat16)
    Wo16 = Wo.astype(jnp.bfloat16)
    out = pl.pallas_call(
        _body,
        out_shape=jax.ShapeDtypeStruct((ROWS, D_MODEL), jnp.float32),
        in_specs=[
            pl.BlockSpec(memory_space=pltpu.VMEM),
            pl.BlockSpec(memory_space=pltpu.VMEM),
            pl.BlockSpec(memory_space=pltpu.VMEM),
            pl.BlockSpec(memory_space=pltpu.VMEM),
            pl.BlockSpec(memory_space=pltpu.VMEM),
        ],
        out_specs=pl.BlockSpec(memory_space=pltpu.VMEM),
        scratch_shapes=[
            pltpu.VMEM((SQ, H_LOC * DH), jnp.float32),
            pltpu.VMEM((960, D_MODEL), jnp.bfloat16),
            pltpu.VMEM((960, D_MODEL), jnp.bfloat16),
            pltpu.VMEM((ROWS, D_MODEL), jnp.bfloat16),
            pltpu.SemaphoreType.DMA((4,)),
            pltpu.SemaphoreType.DMA((4,)),
            pltpu.SemaphoreType.DMA((4,)),
            pltpu.SemaphoreType.DMA((4,)),
        ],
        compiler_params=pltpu.CompilerParams(
            collective_id=0, vmem_limit_bytes=60 * 1024 * 1024),
    )(x16, Wq16, K2, V2, Wo16)
    return out.reshape(B, SQ, D_MODEL)


# device time: 125113 ns/iter; 1.6967x vs baseline; 1.0725x over previous
import jax
import jax.numpy as jnp
from jax import lax
from jax.experimental import pallas as pl
from jax.experimental.pallas import tpu as pltpu

N_DEV = 16
B, SQ, SKV, HQ, DH = 2, 512, 512, 128, 64
H_LOC = HQ // N_DEV
D_MODEL = 768
ROWS = B * SQ
CHUNK = ROWS // N_DEV
BLK = 64

_SHIFT = {1: 0, 2: 1, 4: 2, 8: 3}
_RS_DIMS = [1, 4, 2, 8]
_RS_ROWS = [512, 256, 128, 64]
_RS_OFF = [0, 512, 768, 896]


def _body(x_ref, wq_ref, k_ref, v_ref, wo_ref, out_ref,
          ctx_ref, rs_send, rs_recv, ag_buf,
          rs_send_sems, rs_recv_sems, ag_send_sems, ag_recv_sems):
    my = lax.axis_index("i")

    barrier = pltpu.get_barrier_semaphore()
    for d in _RS_DIMS:
        pl.semaphore_signal(
            barrier, inc=1,
            device_id=(my ^ d,), device_id_type=pl.DeviceIdType.MESH,
        )
    pl.semaphore_wait(barrier, 4)

    qb = lax.broadcasted_iota(jnp.int32, (SQ, SKV), 0) // BLK
    kb = lax.broadcasted_iota(jnp.int32, (SQ, SKV), 1) // BLK
    mask = (qb == kb) | (kb == 0) | (((qb + kb) % 3) == 0)

    bit0 = my & 1
    pending = []

    for j in range(2):
        b = bit0 ^ (1 - j)
        xb = x_ref[pl.ds(b, 1)].reshape(SQ, D_MODEL)
        q_all = jnp.dot(xb, wq_ref[...], preferred_element_type=jnp.float32)
        for h in range(H_LOC):
            q = q_all[:, h * DH:(h + 1) * DH].astype(jnp.bfloat16)
            k = k_ref[pl.ds(b, 1), :, h, :].reshape(SKV, DH)
            v = v_ref[pl.ds(b, 1), :, h, :].reshape(SKV, DH)
            s = lax.dot_general(
                q, k, (((1,), (1,)), ((), ())),
                preferred_element_type=jnp.float32,
            ) * 0.125
            s = jnp.where(mask, s, -1e9)
            m = jnp.max(s, axis=1, keepdims=True)
            w = jnp.exp(s - m)
            w = (w / jnp.sum(w, axis=1, keepdims=True)).astype(jnp.bfloat16)
            ctx_ref[:, h * DH:(h + 1) * DH] = jnp.dot(
                w, v, preferred_element_type=jnp.float32)
        partial = jnp.dot(ctx_ref[...].astype(jnp.bfloat16), wo_ref[...],
                          preferred_element_type=jnp.float32)
        out_ref[pl.ds(b * SQ, SQ), :] = partial
        if j == 0:
            rs_send[pl.ds(0, 512), :] = partial.astype(jnp.bfloat16)
            rdma = pltpu.make_async_remote_copy(
                src_ref=rs_send.at[pl.ds(0, 512), :],
                dst_ref=rs_recv.at[pl.ds(0, 512), :],
                send_sem=rs_send_sems.at[0],
                recv_sem=rs_recv_sems.at[0],
                device_id=(my ^ 1,),
                device_id_type=pl.DeviceIdType.MESH,
            )
            rdma.start()
            pending.append(rdma)
            rdma0 = rdma

    lo = bit0 * 8
    rdma0.wait_recv()
    out_ref[pl.ds(lo * CHUNK, 512), :] = (
        out_ref[pl.ds(lo * CHUNK, 512), :]
        + rs_recv[pl.ds(0, 512), :].astype(jnp.float32))

    sz = 8
    for step in range(1, 4):
        d = _RS_DIMS[step]
        szh = sz // 2
        rows = _RS_ROWS[step]
        off = _RS_OFF[step]
        bit = (my >> _SHIFT[d]) & 1
        send_lo = lo + (1 - bit) * szh
        keep_lo = lo + bit * szh
        rs_send[pl.ds(off, rows), :] = (
            out_ref[pl.ds(send_lo * CHUNK, rows), :].astype(jnp.bfloat16))
        rdma = pltpu.make_async_remote_copy(
            src_ref=rs_send.at[pl.ds(off, rows), :],
            dst_ref=rs_recv.at[pl.ds(off, rows), :],
            send_sem=rs_send_sems.at[step],
            recv_sem=rs_recv_sems.at[step],
            device_id=(my ^ d,),
            device_id_type=pl.DeviceIdType.MESH,
        )
        rdma.start()
        pending.append(rdma)
        rdma.wait_recv()
        out_ref[pl.ds(keep_lo * CHUNK, rows), :] = (
            out_ref[pl.ds(keep_lo * CHUNK, rows), :]
            + rs_recv[pl.ds(off, rows), :].astype(jnp.float32))
        lo = keep_lo
        sz = szh

    ag_buf[pl.ds(lo * CHUNK, CHUNK), :] = (
        out_ref[pl.ds(lo * CHUNK, CHUNK), :].astype(jnp.bfloat16))
    sz = 1
    for step, d in enumerate(reversed(_RS_DIMS)):
        bit = (my >> _SHIFT[d]) & 1
        rows = sz * CHUNK
        rdma = pltpu.make_async_remote_copy(
            src_ref=ag_buf.at[pl.ds(lo * CHUNK, rows), :],
            dst_ref=ag_buf.at[pl.ds(lo * CHUNK, rows), :],
            send_sem=ag_send_sems.at[step],
            recv_sem=ag_recv_sems.at[step],
            device_id=(my ^ d,),
            device_id_type=pl.DeviceIdType.MESH,
        )
        rdma.start()
        pending.append(rdma)
        rdma.wait_recv()
        lo = lo - bit * sz
        sz *= 2

    out_ref[...] = ag_buf[...].astype(jnp.float32)

    for rdma in pending:
        rdma.wait_send()


def kernel(x, Wq, K_ext, V_ext, Wo):
    my = lax.axis_index("i")
    K2 = lax.dynamic_slice_in_dim(K_ext, my * H_LOC, H_LOC, axis=2
                                  ).astype(jnp.bfloat16)
    V2 = lax.dynamic_slice_in_dim(V_ext, my * H_LOC, H_LOC, axis=2
                                  ).astype(jnp.bfloat16)
    x16 = x.astype(jnp.bfloat16)
    Wq16 = Wq.astype(jnp.bfloat16)
    Wo16 = Wo.astype(jnp.bfloat16)
    out = pl.pallas_call(
        _body,
        out_shape=jax.ShapeDtypeStruct((ROWS, D_MODEL), jnp.float32),
        in_specs=[
            pl.BlockSpec(memory_space=pltpu.VMEM),
            pl.BlockSpec(memory_space=pltpu.VMEM),
            pl.BlockSpec(memory_space=pltpu.VMEM),
            pl.BlockSpec(memory_space=pltpu.VMEM),
            pl.BlockSpec(memory_space=pltpu.VMEM),
        ],
        out_specs=pl.BlockSpec(memory_space=pltpu.VMEM),
        scratch_shapes=[
            pltpu.VMEM((SQ, H_LOC * DH), jnp.float32),
            pltpu.VMEM((960, D_MODEL), jnp.bfloat16),
            pltpu.VMEM((960, D_MODEL), jnp.bfloat16),
            pltpu.VMEM((ROWS, D_MODEL), jnp.bfloat16),
            pltpu.SemaphoreType.DMA((4,)),
            pltpu.SemaphoreType.DMA((4,)),
            pltpu.SemaphoreType.DMA((4,)),
            pltpu.SemaphoreType.DMA((4,)),
        ],
        compiler_params=pltpu.CompilerParams(
            collective_id=0, vmem_limit_bytes=60 * 1024 * 1024),
    )(x16, Wq16, K2, V2, Wo16)
    return out.reshape(B, SQ, D_MODEL)


# device time: 124018 ns/iter; 1.7117x vs baseline; 1.0088x over previous
import jax
import jax.numpy as jnp
from jax import lax
from jax.experimental import pallas as pl
from jax.experimental.pallas import tpu as pltpu

N_DEV = 16
B, SQ, SKV, HQ, DH = 2, 512, 512, 128, 64
H_LOC = HQ // N_DEV
D_MODEL = 768
ROWS = B * SQ
CHUNK = ROWS // N_DEV
BLK = 64

_SHIFT = {1: 0, 2: 1, 4: 2, 8: 3}
_RS_DIMS = [1, 4, 2, 8]
_RS_ROWS = [512, 256, 128, 64]
_RS_OFF = [0, 512, 768, 896]


def _body(x_ref, wq_ref, k_hbm, v_hbm, wo_ref, out_ref,
          k_ref, v_ref, acc_ref, ctx_ref, rs_send, rs_recv, ag_buf,
          kv_sems, rs_send_sems, rs_recv_sems, ag_send_sems, ag_recv_sems):
    my = lax.axis_index("i")

    kcp = pltpu.make_async_copy(k_hbm, k_ref, kv_sems.at[0])
    vcp = pltpu.make_async_copy(v_hbm, v_ref, kv_sems.at[1])
    kcp.start()
    vcp.start()

    barrier = pltpu.get_barrier_semaphore()
    for d in _RS_DIMS:
        pl.semaphore_signal(
            barrier, inc=1,
            device_id=(my ^ d,), device_id_type=pl.DeviceIdType.MESH,
        )
    kcp.wait()
    vcp.wait()
    pl.semaphore_wait(barrier, 4)

    qb = lax.broadcasted_iota(jnp.int32, (SQ, SKV), 0) // BLK
    kb = lax.broadcasted_iota(jnp.int32, (SQ, SKV), 1) // BLK
    mask = (qb == kb) | (kb == 0) | (((qb + kb) % 3) == 0)

    bit0 = my & 1
    pending = []

    for j in range(2):
        b = bit0 ^ (1 - j)
        xb = x_ref[pl.ds(b, 1)].reshape(SQ, D_MODEL)
        q_all = jnp.dot(xb, wq_ref[...], preferred_element_type=jnp.float32)
        for h in range(H_LOC):
            q = q_all[:, h * DH:(h + 1) * DH].astype(jnp.bfloat16)
            k = k_ref[pl.ds(b, 1), :, h, :].reshape(SKV, DH)
            v = v_ref[pl.ds(b, 1), :, h, :].reshape(SKV, DH)
            s = lax.dot_general(
                q, k, (((1,), (1,)), ((), ())),
                preferred_element_type=jnp.float32,
            ) * 0.125
            s = jnp.where(mask, s, -1e9)
            m = jnp.max(s, axis=1, keepdims=True)
            w = jnp.exp(s - m)
            w = (w / jnp.sum(w, axis=1, keepdims=True)).astype(jnp.bfloat16)
            ctx_ref[:, h * DH:(h + 1) * DH] = jnp.dot(
                w, v, preferred_element_type=jnp.float32)
        partial = jnp.dot(ctx_ref[...].astype(jnp.bfloat16), wo_ref[...],
                          preferred_element_type=jnp.float32)
        acc_ref[pl.ds(b * SQ, SQ), :] = partial
        if j == 0:
            rs_send[pl.ds(0, 512), :] = partial.astype(jnp.bfloat16)
            rdma = pltpu.make_async_remote_copy(
                src_ref=rs_send.at[pl.ds(0, 512), :],
                dst_ref=rs_recv.at[pl.ds(0, 512), :],
                send_sem=rs_send_sems.at[0],
                recv_sem=rs_recv_sems.at[0],
                device_id=(my ^ 1,),
                device_id_type=pl.DeviceIdType.MESH,
            )
            rdma.start()
            pending.append(rdma)
            rdma0 = rdma

    lo = bit0 * 8
    rdma0.wait_recv()
    acc_ref[pl.ds(lo * CHUNK, 512), :] = (
        acc_ref[pl.ds(lo * CHUNK, 512), :]
        + rs_recv[pl.ds(0, 512), :].astype(jnp.float32))

    sz = 8
    for step in range(1, 4):
        d = _RS_DIMS[step]
        szh = sz // 2
        rows = _RS_ROWS[step]
        off = _RS_OFF[step]
        bit = (my >> _SHIFT[d]) & 1
        send_lo = lo + (1 - bit) * szh
        keep_lo = lo + bit * szh
        rs_send[pl.ds(off, rows), :] = (
            acc_ref[pl.ds(send_lo * CHUNK, rows), :].astype(jnp.bfloat16))
        rdma = pltpu.make_async_remote_copy(
            src_ref=rs_send.at[pl.ds(off, rows), :],
            dst_ref=rs_recv.at[pl.ds(off, rows), :],
            send_sem=rs_send_sems.at[step],
            recv_sem=rs_recv_sems.at[step],
            device_id=(my ^ d,),
            device_id_type=pl.DeviceIdType.MESH,
        )
        rdma.start()
        pending.append(rdma)
        rdma.wait_recv()
        acc_ref[pl.ds(keep_lo * CHUNK, rows), :] = (
            acc_ref[pl.ds(keep_lo * CHUNK, rows), :]
            + rs_recv[pl.ds(off, rows), :].astype(jnp.float32))
        lo = keep_lo
        sz = szh

    ag_buf[pl.ds(lo * CHUNK, CHUNK), :] = (
        acc_ref[pl.ds(lo * CHUNK, CHUNK), :].astype(jnp.bfloat16))
    sz = 1
    for step, d in enumerate(reversed(_RS_DIMS)):
        bit = (my >> _SHIFT[d]) & 1
        rows = sz * CHUNK
        rdma = pltpu.make_async_remote_copy(
            src_ref=ag_buf.at[pl.ds(lo * CHUNK, rows), :],
            dst_ref=ag_buf.at[pl.ds(lo * CHUNK, rows), :],
            send_sem=ag_send_sems.at[step],
            recv_sem=ag_recv_sems.at[step],
            device_id=(my ^ d,),
            device_id_type=pl.DeviceIdType.MESH,
        )
        rdma.start()
        pending.append(rdma)
        rdma.wait_recv()
        lo = lo - bit * sz
        sz *= 2

    out_ref[0] = ag_buf[pl.ds(0, SQ), :].astype(jnp.float32)
    out_ref[1] = ag_buf[pl.ds(SQ, SQ), :].astype(jnp.float32)

    for rdma in pending:
        rdma.wait_send()


def kernel(x, Wq, K_ext, V_ext, Wo):
    my = lax.axis_index("i")
    K2 = lax.dynamic_slice_in_dim(K_ext, my * H_LOC, H_LOC, axis=2
                                  ).astype(jnp.bfloat16)
    V2 = lax.dynamic_slice_in_dim(V_ext, my * H_LOC, H_LOC, axis=2
                                  ).astype(jnp.bfloat16)
    x16 = x.astype(jnp.bfloat16)
    Wq16 = Wq.astype(jnp.bfloat16)
    Wo16 = Wo.astype(jnp.bfloat16)
    out = pl.pallas_call(
        _body,
        out_shape=jax.ShapeDtypeStruct((B, SQ, D_MODEL), jnp.float32),
        in_specs=[
            pl.BlockSpec(memory_space=pltpu.VMEM),
            pl.BlockSpec(memory_space=pltpu.VMEM),
            pl.BlockSpec(memory_space=pltpu.MemorySpace.HBM),
            pl.BlockSpec(memory_space=pltpu.MemorySpace.HBM),
            pl.BlockSpec(memory_space=pltpu.VMEM),
        ],
        out_specs=pl.BlockSpec(memory_space=pltpu.VMEM),
        scratch_shapes=[
            pltpu.VMEM((B, SKV, H_LOC, DH), jnp.bfloat16),
            pltpu.VMEM((B, SKV, H_LOC, DH), jnp.bfloat16),
            pltpu.VMEM((ROWS, D_MODEL), jnp.float32),
            pltpu.VMEM((SQ, H_LOC * DH), jnp.float32),
            pltpu.VMEM((960, D_MODEL), jnp.bfloat16),
            pltpu.VMEM((960, D_MODEL), jnp.bfloat16),
            pltpu.VMEM((ROWS, D_MODEL), jnp.bfloat16),
            pltpu.SemaphoreType.DMA((2,)),
            pltpu.SemaphoreType.DMA((4,)),
            pltpu.SemaphoreType.DMA((4,)),
            pltpu.SemaphoreType.DMA((4,)),
            pltpu.SemaphoreType.DMA((4,)),
        ],
        compiler_params=pltpu.CompilerParams(
            collective_id=0, vmem_limit_bytes=60 * 1024 * 1024),
    )(x16, Wq16, K2, V2, Wo16)
    return out
